# baseline (speedup 1.0000x reference)
import functools

import jax
import jax.numpy as jnp
from jax import lax
from jax.experimental import pallas as pl
from jax.experimental.pallas import tpu as pltpu
from jax.experimental.pallas import tpu_sc as plsc

N = 10000
E = 320000
D = 128
H = 128
OUTD = 128
G = 64
NEG = 0.2
ACT_NEG = 0.01

NP = 10112
NT = 32
NB = NP // 128
NR = 128
NCH = 80
CH = 128
EPT = NCH * CH
EP = NT * EPT
RPT = NP // 16
EC = 64
NC2 = EPT // EC
WCH = 8
NP2 = 10240
RD = NP2 // 16

_MESH = plsc.VectorSubcoreMesh(
    core_axis_name="c", subcore_axis_name="s", num_cores=2, num_subcores=16)
_SC_PARAMS = pltpu.CompilerParams(needs_layout_passes=False)


def _zero_1d(ref, n):
    z16 = jnp.zeros((16,), jnp.float32)

    def zb(r, carry):
        ref[pl.ds(r * 16, 16)] = z16
        return carry

    lax.fori_loop(0, n // 16, zb, 0, unroll=8)


def _layer_a_body(hin, w, avec, we, ae, h_o, asrc_o, adst_o, pck_o, consts_o,
                  macc):
    i = pl.program_id(0)
    h = jnp.dot(hin[...], w[...], preferred_element_type=jnp.float32)
    proj = jnp.dot(h, avec[...].T, preferred_element_type=jnp.float32)
    asr = proj[:, 0]
    adt = proj[:, 1]
    h_o[...] = h
    asrc_o[...] = asr[None, None, :]
    adst_o[...] = adt[None, None, :]
    ai = lax.bitcast_convert_type(asr, jnp.int32)
    di = lax.bitcast_convert_type(adt, jnp.int32)
    ai = (ai + 0x8000) & jnp.int32(-65536)
    di = lax.shift_right_logical((di + 0x8000) & jnp.int32(-65536), 16)
    pck_o[...] = (ai | di)[None, None, :]

    @pl.when(i == 0)
    def _():
        macc[0] = jnp.float32(-1e30)
        macc[1] = jnp.float32(-1e30)

    macc[0] = jnp.maximum(macc[0], jnp.max(asr))
    macc[1] = jnp.maximum(macc[1], jnp.max(adt))
    w3 = jnp.sum(we[...] * ae[...], axis=1)
    ub_e = jnp.sum(jnp.maximum(w3, 0.0))
    c = jnp.maximum(macc[0] + macc[1] + ub_e, 0.0)
    row = lax.broadcasted_iota(jnp.int32, (8, 128), 0)
    consts_o[...] = jnp.where(
        row == 0, w3[0], jnp.where(row == 1, w3[1], jnp.where(row == 2, w3[2], c)))


def _layer_a(hin, w, avec, we, ae1):
    return pl.pallas_call(
        _layer_a_body,
        grid=(NB,),
        in_specs=[
            pl.BlockSpec((NR, D), lambda i: (i, 0)),
            pl.BlockSpec((D, H), lambda i: (0, 0)),
            pl.BlockSpec((2, H), lambda i: (0, 0)),
            pl.BlockSpec((3, H), lambda i: (0, 0)),
            pl.BlockSpec((1, H), lambda i: (0, 0)),
        ],
        out_specs=[
            pl.BlockSpec((NR, H), lambda i: (i, 0)),
            pl.BlockSpec((1, 1, NR), lambda i: (i, 0, 0)),
            pl.BlockSpec((1, 1, NR), lambda i: (i, 0, 0)),
            pl.BlockSpec((1, 1, NR), lambda i: (i, 0, 0)),
            pl.BlockSpec((8, 128), lambda i: (0, 0)),
        ],
        out_shape=[
            jax.ShapeDtypeStruct((NP, H), jnp.float32),
            jax.ShapeDtypeStruct((NB, 1, NR), jnp.float32),
            jax.ShapeDtypeStruct((NB, 1, NR), jnp.float32),
            jax.ShapeDtypeStruct((NB, 1, NR), jnp.int32),
            jax.ShapeDtypeStruct((8, 128), jnp.float32),
        ],
        scratch_shapes=[pltpu.SMEM((2,), jnp.float32)],
    )(hin, w, avec, we, ae1)


def _aedge_body(ea, dst, we, ae, out):
    w3 = jnp.sum(we[...] * ae[...], axis=1)
    aev = ea[0] * w3[0] + ea[1] * w3[1] + ea[2] * w3[2]
    bits = (lax.bitcast_convert_type(aev, jnp.int32) + 0x8000) & jnp.int32(
        -65536)
    out[...] = bits | dst[...]


def _aedge(ea_r, dst_r, we, ae1):
    nblk = EP // (8 * 128)
    return pl.pallas_call(
        _aedge_body,
        grid=(nblk,),
        in_specs=[
            pl.BlockSpec((3, 8, 128), lambda i: (0, i, 0)),
            pl.BlockSpec((8, 128), lambda i: (i, 0)),
            pl.BlockSpec((3, H), lambda i: (0, 0)),
            pl.BlockSpec((1, H), lambda i: (0, 0)),
        ],
        out_specs=pl.BlockSpec((8, 128), lambda i: (i, 0)),
        out_shape=jax.ShapeDtypeStruct((EP // 128, 128), jnp.int32),
    )(ea_r, dst_r, we, ae1)


@functools.partial(
    pl.kernel,
    mesh=_MESH,
    out_type=jax.ShapeDtypeStruct((2 * 4 * NP2,), jnp.float32),
    scratch_types=[
        pltpu.VMEM((RD,), jnp.float32),
        pltpu.VMEM((RD,), jnp.float32),
        pltpu.VMEM_SHARED((16 * NP2,), jnp.float32),
    ],
    compiler_params=_SC_PARAMS,
)
def _phase0(dst_hbm, ea_hbm, dm_hbm, acc, stg, part_sp):
    cid = lax.axis_index("c")
    sid = lax.axis_index("s")
    wid = cid * 16 + sid

    def body(dst_v, ea0_v, ea1_v, ea2_v, tab):
        _zero_1d(tab, 4 * NP2)

        ones = jnp.ones((16,), jnp.float32)

        def window(wi, carry):
            w0 = pl.multiple_of(wi * 8, 8)
            pltpu.sync_copy(dst_hbm.at[wid, pl.ds(w0, 8)], dst_v)
            pltpu.sync_copy(ea_hbm.at[0, wid, pl.ds(w0, 8)], ea0_v)
            pltpu.sync_copy(ea_hbm.at[1, wid, pl.ds(w0, 8)], ea1_v)
            pltpu.sync_copy(ea_hbm.at[2, wid, pl.ds(w0, 8)], ea2_v)
            for jj in range(8):
                for g in range(CH // 16):
                    sl = pl.ds(g * 16, 16)
                    didx = dst_v[jj, sl]
                    plsc.addupdate_scatter(tab, [didx], ones)
                    plsc.addupdate_scatter(tab, [didx + NP2], ea0_v[jj, sl])
                    plsc.addupdate_scatter(
                        tab, [didx + 2 * NP2], ea1_v[jj, sl])
                    plsc.addupdate_scatter(
                        tab, [didx + 3 * NP2], ea2_v[jj, sl])
            return carry

        lax.fori_loop(0, NCH // 8, window, 0)

        for c in range(4):
            pltpu.sync_copy(tab.at[pl.ds(c * NP2, NP2)],
                            part_sp.at[pl.ds(sid * NP2, NP2)])
            plsc.subcore_barrier()
            pltpu.sync_copy(part_sp.at[pl.ds(sid * RD, RD)], acc)
            for t in range(1, 16):
                pltpu.sync_copy(part_sp.at[pl.ds(t * NP2 + sid * RD, RD)], stg)

                def addb(r, carry):
                    sl = pl.ds(r * 16, 16)
                    acc[sl] = acc[sl] + stg[sl]
                    return carry

                lax.fori_loop(0, RD // 16, addb, 0, unroll=8)
            pltpu.sync_copy(
                acc, dm_hbm.at[pl.ds(cid * 4 * NP2 + c * NP2 + sid * RD, RD)])
            plsc.subcore_barrier()

    pl.run_scoped(
        body,
        pltpu.VMEM((8, CH), jnp.int32),
        pltpu.VMEM((8, CH), jnp.float32),
        pltpu.VMEM((8, CH), jnp.float32),
        pltpu.VMEM((8, CH), jnp.float32),
        pltpu.VMEM((4 * NP2,), jnp.float32),
    )


@functools.partial(
    pl.kernel,
    mesh=_MESH,
    out_type=[
        jax.ShapeDtypeStruct((2, NP, H), jnp.float32),
        jax.ShapeDtypeStruct((2 * NP2,), jnp.float32),
    ],
    scratch_types=[
        pltpu.VMEM((128,), jnp.float32),
        pltpu.VMEM((128,), jnp.float32),
        pltpu.VMEM((128,), jnp.float32),
        pltpu.VMEM_SHARED((NP, H), jnp.float32),
        pltpu.VMEM_SHARED((16 * NP2,), jnp.float32),
        pltpu.SemaphoreType.DMA,
        pltpu.SemaphoreType.DMA,
    ],
    compiler_params=_SC_PARAMS,
)
def _edge(h_hbm, pck_hbm, consts_hbm, src_hbm, dstae_hbm,
          num_hbm, den_hbm, alp_v, acc, stg, num_sp, dpart_sp, gsem, ssem):
    cid = lax.axis_index("c")
    sid = lax.axis_index("s")
    wid = cid * 16 + sid

    def body(pck_v, consts_v, srcw, daw, cidx0, cidx1, rows0, rows1, den_v):
        pltpu.sync_copy(pck_hbm, pck_v)
        pltpu.sync_copy(consts_hbm.at[3], consts_v)

        _zero_1d(den_v, NP)
        z16 = jnp.zeros((16,), jnp.float32)

        def zrow(r, carry):
            for c8 in range(H // 16):
                rows0[r, pl.ds(c8 * 16, 16)] = z16
            return carry

        lax.fori_loop(0, EC, zrow, 0)
        r0 = sid * RPT
        for k in range(RPT // EC):
            pltpu.sync_copy(rows0, num_sp.at[pl.ds(r0 + k * EC, EC)])
        rem = RPT - (RPT // EC) * EC
        if rem:
            pltpu.sync_copy(rows0.at[pl.ds(0, rem)],
                            num_sp.at[pl.ds(r0 + RPT - rem, rem)])
        plsc.subcore_barrier()

        cvec = consts_v[pl.ds(48, 16)]
        mhi = jnp.full((16,), -65536, jnp.int32)
        mlo = jnp.full((16,), 65535, jnp.int32)

        def alpha_scale(jj, cidxb, rowsb):
            for g in range(EC // 16):
                sl16 = pl.ds(g * 16, 16)
                da = daw[jj, sl16]
                didx = da & mlo
                cidxb[sl16] = didx
                ps = plsc.load_gather(pck_v, [srcw[jj, sl16]])
                pd = plsc.load_gather(pck_v, [didx])
                a_s = plsc.bitcast(ps & mhi, jnp.float32)
                a_d = plsc.bitcast(lax.shift_left(pd, 16), jnp.float32)
                a = a_s + a_d + plsc.bitcast(da & mhi, jnp.float32)
                a = jnp.where(a >= 0, a, NEG * a)
                al = jnp.exp(a - cvec)
                alp_v[sl16] = al
                plsc.addupdate_scatter(den_v, [didx], al)

            def sgrp(g2, carry2):
                av = alp_v[pl.ds(g2 * 16, 16)]
                for k in range(16):
                    s = av[k]
                    r = g2 * 16 + k
                    for c8 in range(H // 16):
                        csl = pl.ds(c8 * 16, 16)
                        rowsb[r, csl] = rowsb[r, csl] * s
                return carry2

            lax.fori_loop(0, EC // 16, sgrp, 0)

        def wait_gather(jj, rowsb):
            pltpu.make_async_copy(h_hbm.at[srcw.at[jj]], rowsb, gsem).wait()

        def wait_scatter(rowsb, cidxb):
            pltpu.make_async_copy(rowsb, num_sp.at[cidxb], ssem).wait()

        def window(wi, carry):
            w0 = pl.multiple_of(wi * WCH, 8)
            pltpu.sync_copy(src_hbm.at[wid, pl.ds(w0, WCH)], srcw)
            pltpu.sync_copy(dstae_hbm.at[wid, pl.ds(w0, WCH)], daw)
            pltpu.async_copy(h_hbm.at[srcw.at[0]], rows0, gsem)

            def pair(t, c2):
                a = 2 * t
                b = a + 1

                @pl.when(t > 0)
                def _():
                    wait_scatter(rows1, cidx1)

                pltpu.async_copy(h_hbm.at[srcw.at[b]], rows1, gsem)
                wait_gather(a, rows0)
                alpha_scale(a, cidx0, rows0)
                pltpu.async_copy(rows0, num_sp.at[cidx0], ssem, add=True)
                wait_gather(b, rows1)
                alpha_scale(b, cidx1, rows1)
                wait_scatter(rows0, cidx0)

                @pl.when(t < WCH // 2 - 1)
                def _():
                    pltpu.async_copy(h_hbm.at[srcw.at[a + 2]], rows0, gsem)

                pltpu.async_copy(rows1, num_sp.at[cidx1], ssem, add=True)
                return c2

            lax.fori_loop(0, WCH // 2, pair, 0)
            wait_scatter(rows1, cidx1)
            return carry

        lax.fori_loop(0, NC2 // WCH, window, 0)
        pltpu.sync_copy(den_v, dpart_sp.at[pl.ds(sid * NP2, NP)])
        plsc.subcore_barrier()

        pltpu.sync_copy(num_sp.at[pl.ds(r0, RPT)],
                        num_hbm.at[cid, pl.ds(r0, RPT)])
        for b in range(RD // 128):
            o = sid * RD + b * 128
            pltpu.sync_copy(dpart_sp.at[pl.ds(o, 128)], acc)
            for t in range(1, 16):
                pltpu.sync_copy(dpart_sp.at[pl.ds(t * NP2 + o, 128)], stg)
                for r in range(8):
                    sl = pl.ds(r * 16, 16)
                    acc[sl] = acc[sl] + stg[sl]
            pltpu.sync_copy(acc, den_hbm.at[pl.ds(cid * NP2 + o, 128)])

    pl.run_scoped(
        body,
        pltpu.VMEM((NP,), jnp.int32),
        pltpu.VMEM((128,), jnp.float32),
        pltpu.VMEM((WCH, EC), jnp.int32),
        pltpu.VMEM((WCH, EC), jnp.int32),
        pltpu.VMEM((EC,), jnp.int32),
        pltpu.VMEM((EC,), jnp.int32),
        pltpu.VMEM((EC, H), jnp.float32),
        pltpu.VMEM((EC, H), jnp.float32),
        pltpu.VMEM((NP,), jnp.float32),
    )


def _epi_body(num, h_in, asrc, adst, den_in, dm, consts, bgb, hout):
    nsum = num[0] + num[1]
    den = den_in[0, 0] + den_in[0, 1]
    h = h_in[...]
    deg = jnp.maximum(dm[0, 0, 0] + dm[0, 1, 0], 1.0)
    w3_0 = consts[0, 0]
    w3_1 = consts[1, 0]
    w3_2 = consts[2, 0]
    c = consts[3, 0]
    mlog = ((dm[0, 0, 1] + dm[0, 1, 1]) * w3_0
            + (dm[0, 0, 2] + dm[0, 1, 2]) * w3_1
            + (dm[0, 0, 3] + dm[0, 1, 3]) * w3_2) / deg
    logit = asrc[0, 0] + adst[0, 0] + mlog
    logit = jnp.where(logit >= 0, logit, NEG * logit)
    al = jnp.exp(logit - c)
    nsum = nsum + al[:, None] * h
    den = jnp.maximum(den + al, 1e-16)
    out = nsum / den[:, None] + bgb[0][None, :]
    out = jnp.where(out >= 0, out, ACT_NEG * out)
    m = jnp.mean(out, axis=1, keepdims=True)
    v = jnp.mean((out - m) ** 2, axis=1, keepdims=True)
    out_ln = (out - m) / jnp.sqrt(v + 1e-5) * bgb[1][None, :] + bgb[2][None, :]
    hout[...] = jnp.where(bgb[3, 0] > 0.5, out_ln, out)


def _epilogue(num, h, asrc, adst, den, dm, consts, bgb):
    return pl.pallas_call(
        _epi_body,
        grid=(NB,),
        in_specs=[
            pl.BlockSpec((2, NR, H), lambda i: (0, i, 0)),
            pl.BlockSpec((NR, H), lambda i: (i, 0)),
            pl.BlockSpec((1, 1, NR), lambda i: (i, 0, 0)),
            pl.BlockSpec((1, 1, NR), lambda i: (i, 0, 0)),
            pl.BlockSpec((1, 2, NR), lambda i: (i, 0, 0)),
            pl.BlockSpec((1, 2, 4, NR), lambda i: (i, 0, 0, 0)),
            pl.BlockSpec((8, 128), lambda i: (0, 0)),
            pl.BlockSpec((4, H), lambda i: (0, 0)),
        ],
        out_specs=pl.BlockSpec((NR, D), lambda i: (i, 0)),
        out_shape=jax.ShapeDtypeStruct((NP, D), jnp.float32),
    )(num, h, asrc, adst, den, dm, consts, bgb)


def _head_body(h, wm1, bm1, wm2, bm2, batchb, out, acc, cnt):
    i = pl.program_id(0)

    @pl.when(i == 0)
    def _():
        acc[...] = jnp.zeros_like(acc)
        cnt[...] = jnp.zeros_like(cnt)

    h2 = jnp.dot(h[...], wm1[...], preferred_element_type=jnp.float32) + bm1[...]
    h2 = jnp.dot(h2, wm2[...], preferred_element_type=jnp.float32) + bm2[...]
    b = batchb[0, 0]
    oh = (lax.broadcasted_iota(jnp.int32, (G, NR), 0) == b[None, :]).astype(
        jnp.float32)
    acc[...] = acc[...] + jnp.dot(oh, h2, preferred_element_type=jnp.float32)
    cnt[...] = cnt[...] + jnp.sum(oh, axis=1, keepdims=True)

    @pl.when(i == NB - 1)
    def _():
        out[...] = acc[...] / jnp.maximum(cnt[...], 1.0)


def _head(h, wm1, bm1, wm2, bm2, batchb):
    return pl.pallas_call(
        _head_body,
        grid=(NB,),
        in_specs=[
            pl.BlockSpec((NR, D), lambda i: (i, 0)),
            pl.BlockSpec((H, OUTD // 2), lambda i: (0, 0)),
            pl.BlockSpec((1, OUTD // 2), lambda i: (0, 0)),
            pl.BlockSpec((OUTD // 2, OUTD), lambda i: (0, 0)),
            pl.BlockSpec((1, OUTD), lambda i: (0, 0)),
            pl.BlockSpec((1, 1, NR), lambda i: (i, 0, 0)),
        ],
        out_specs=pl.BlockSpec((G, OUTD), lambda i: (0, 0)),
        out_shape=jax.ShapeDtypeStruct((G, OUTD), jnp.float32),
        scratch_shapes=[
            pltpu.VMEM((G, OUTD), jnp.float32),
            pltpu.VMEM((G, 128), jnp.float32),
        ],
    )(h, wm1, bm1, wm2, bm2, batchb)


def kernel(x, edge_index, edge_attr, batch, W0, as0, ad0, We0, ae0, b0,
           W1, as1, ad1, We1, ae1, b1, W2, as2, ad2, We2, ae2, b2,
           g0, be0, g1, be1, Wm1, bm1, Wm2, bm2):
    f32 = jnp.float32
    i32 = jnp.int32
    xp = jnp.zeros((NP, D), f32).at[:N, :].set(x)
    src = edge_index[0]
    dst = edge_index[1]
    padi = jnp.full((EP - E,), NP - 1, i32)
    src3 = jnp.concatenate([src, padi]).reshape(NT, NCH, CH)
    dst3 = jnp.concatenate([dst, padi]).reshape(NT, NCH, CH)
    ea_t = jnp.concatenate(
        [edge_attr, jnp.zeros((EP - E, 3), f32)], axis=0).T.reshape(
            3, NT, NCH, CH)
    batch3 = jnp.concatenate(
        [batch, jnp.full((NP - N,), G, i32)]).reshape(NB, 1, NR)

    dm = _phase0(dst3, ea_t).reshape(2, 4, NP2)[:, :, :NP].reshape(
        2, 4, NB, NR).transpose(2, 0, 1, 3)

    zeros_h = jnp.zeros((H,), f32)
    ones_h = jnp.ones((H,), f32)
    w_all = jnp.stack([W0, W1, W2])
    avec_all = jnp.stack([jnp.stack([as0, ad0]), jnp.stack([as1, ad1]),
                          jnp.stack([as2, ad2])])
    we_all = jnp.stack([We0, We1, We2])
    ae_all = jnp.stack([ae0.reshape(1, H), ae1.reshape(1, H),
                        ae2.reshape(1, H)])
    bgb_all = jnp.stack([
        jnp.stack([b0, g0, be0, ones_h]),
        jnp.stack([b1, g1, be1, ones_h]),
        jnp.stack([b2, zeros_h, zeros_h, zeros_h]),
    ])

    ea_r = ea_t.reshape(3, EP // 128, 128)

    dst_r = dst3.reshape(EP // 128, 128)

    src_e = src3.reshape(NT, NC2, EC)

    def step(h, ws):
        w, avec, we, ae1, bgb = ws
        h_l, asrc, adst, pck, consts = _layer_a(h, w, avec, we, ae1)
        dstae = _aedge(ea_r, dst_r, we, ae1).reshape(NT, NC2, EC)
        num, den = _edge(h_l, pck.reshape(NP), consts, src_e, dstae)
        den = den.reshape(2, NP2)[:, :NP].reshape(
            2, NB, NR).transpose(1, 0, 2)
        h_next = _epilogue(num, h_l, asrc, adst, den, dm, consts, bgb)
        return h_next, None

    h, _ = lax.scan(step, xp, (w_all, avec_all, we_all, ae_all, bgb_all))

    return _head(h, Wm1, bm1.reshape(1, OUTD // 2), Wm2,
                 bm2.reshape(1, OUTD), batch3)


# speedup vs baseline: 1.0658x; 1.0658x over previous
import functools

import jax
import jax.numpy as jnp
from jax import lax
from jax.experimental import pallas as pl
from jax.experimental.pallas import tpu as pltpu
from jax.experimental.pallas import tpu_sc as plsc

N = 10000
E = 320000
D = 128
H = 128
OUTD = 128
G = 64
NEG = 0.2
ACT_NEG = 0.01

NP = 10112
NT = 32
NB = NP // 128
NR = 128
NCH = 80
CH = 128
EPT = NCH * CH
EP = NT * EPT
RPT = NP // 16
NP2 = 10240
RD = NP2 // 16

_MESH = plsc.VectorSubcoreMesh(
    core_axis_name="c", subcore_axis_name="s", num_cores=2, num_subcores=16)
_SC_PARAMS = pltpu.CompilerParams(
    needs_layout_passes=False, skip_device_barrier=True,
    disable_bounds_checks=True, disable_semaphore_checks=True)


def _zero_1d(ref, n):
    z16 = jnp.zeros((16,), jnp.float32)

    def zb(r, carry):
        ref[pl.ds(r * 16, 16)] = z16
        return carry

    lax.fori_loop(0, n // 16, zb, 0, unroll=8)


def _layer_a_body(hin, w, avec, we, ae, h_o, asrc_o, adst_o, pck_o, consts_o,
                  macc):
    i = pl.program_id(0)
    h = jnp.dot(hin[...], w[...], preferred_element_type=jnp.float32)
    proj = jnp.dot(h, avec[...].T, preferred_element_type=jnp.float32)
    asr = proj[:, 0]
    adt = proj[:, 1]
    h_o[...] = h
    asrc_o[...] = asr[None, None, :]
    adst_o[...] = adt[None, None, :]
    ai = lax.bitcast_convert_type(asr, jnp.int32)
    di = lax.bitcast_convert_type(adt, jnp.int32)
    ai = (ai + 0x8000) & jnp.int32(-65536)
    di = lax.shift_right_logical((di + 0x8000) & jnp.int32(-65536), 16)
    pck_o[...] = (ai | di)[None, None, :]

    @pl.when(i == 0)
    def _():
        macc[0] = jnp.float32(-1e30)
        macc[1] = jnp.float32(-1e30)

    macc[0] = jnp.maximum(macc[0], jnp.max(asr))
    macc[1] = jnp.maximum(macc[1], jnp.max(adt))
    w3 = jnp.sum(we[...] * ae[...], axis=1)
    ub_e = jnp.sum(jnp.maximum(w3, 0.0))
    c = jnp.maximum(macc[0] + macc[1] + ub_e, 0.0)
    row = lax.broadcasted_iota(jnp.int32, (8, 128), 0)
    consts_o[...] = jnp.where(
        row == 0, w3[0], jnp.where(row == 1, w3[1], jnp.where(row == 2, w3[2], c)))


def _layer_a(hin, w, avec, we, ae1):
    return pl.pallas_call(
        _layer_a_body,
        grid=(NB,),
        in_specs=[
            pl.BlockSpec((NR, D), lambda i: (i, 0)),
            pl.BlockSpec((D, H), lambda i: (0, 0)),
            pl.BlockSpec((2, H), lambda i: (0, 0)),
            pl.BlockSpec((3, H), lambda i: (0, 0)),
            pl.BlockSpec((1, H), lambda i: (0, 0)),
        ],
        out_specs=[
            pl.BlockSpec((NR, H), lambda i: (i, 0)),
            pl.BlockSpec((1, 1, NR), lambda i: (i, 0, 0)),
            pl.BlockSpec((1, 1, NR), lambda i: (i, 0, 0)),
            pl.BlockSpec((1, 1, NR), lambda i: (i, 0, 0)),
            pl.BlockSpec((8, 128), lambda i: (0, 0)),
        ],
        out_shape=[
            jax.ShapeDtypeStruct((NP, H), jnp.float32),
            jax.ShapeDtypeStruct((NB, 1, NR), jnp.float32),
            jax.ShapeDtypeStruct((NB, 1, NR), jnp.float32),
            jax.ShapeDtypeStruct((NB, 1, NR), jnp.int32),
            jax.ShapeDtypeStruct((8, 128), jnp.float32),
        ],
        scratch_shapes=[pltpu.SMEM((2,), jnp.float32)],
    )(hin, w, avec, we, ae1)


def _aedge_body(ea, dst, we, ae, out):
    w3 = jnp.sum(we[...] * ae[...], axis=1)
    aev = ea[0] * w3[0] + ea[1] * w3[1] + ea[2] * w3[2]
    bits = (lax.bitcast_convert_type(aev, jnp.int32) + 0x8000) & jnp.int32(
        -65536)
    out[...] = bits | dst[...]


def _aedge(ea_r, dst_r, we, ae1):
    nblk = EP // (8 * 128)
    return pl.pallas_call(
        _aedge_body,
        grid=(nblk,),
        in_specs=[
            pl.BlockSpec((3, 8, 128), lambda i: (0, i, 0)),
            pl.BlockSpec((8, 128), lambda i: (i, 0)),
            pl.BlockSpec((3, H), lambda i: (0, 0)),
            pl.BlockSpec((1, H), lambda i: (0, 0)),
        ],
        out_specs=pl.BlockSpec((8, 128), lambda i: (i, 0)),
        out_shape=jax.ShapeDtypeStruct((EP // 128, 128), jnp.int32),
    )(ea_r, dst_r, we, ae1)


@functools.partial(
    pl.kernel,
    mesh=_MESH,
    out_type=jax.ShapeDtypeStruct((2 * 4 * NP2,), jnp.float32),
    scratch_types=[
        pltpu.VMEM((RD,), jnp.float32),
        pltpu.VMEM((RD,), jnp.float32),
        pltpu.VMEM_SHARED((16 * NP2,), jnp.float32),
    ],
    compiler_params=_SC_PARAMS,
)
def _phase0(dst_hbm, ea_hbm, dm_hbm, acc, stg, part_sp):
    cid = lax.axis_index("c")
    sid = lax.axis_index("s")
    wid = cid * 16 + sid

    def body(dst_v, ea0_v, ea1_v, ea2_v, tab):
        _zero_1d(tab, 4 * NP2)

        ones = jnp.ones((16,), jnp.float32)

        def window(wi, carry):
            w0 = pl.multiple_of(wi * 8, 8)
            pltpu.sync_copy(dst_hbm.at[wid, pl.ds(w0, 8)], dst_v)
            pltpu.sync_copy(ea_hbm.at[0, wid, pl.ds(w0, 8)], ea0_v)
            pltpu.sync_copy(ea_hbm.at[1, wid, pl.ds(w0, 8)], ea1_v)
            pltpu.sync_copy(ea_hbm.at[2, wid, pl.ds(w0, 8)], ea2_v)
            for jj in range(8):
                for g in range(CH // 16):
                    sl = pl.ds(g * 16, 16)
                    didx = dst_v[jj, sl]
                    plsc.addupdate_scatter(tab, [didx], ones)
                    plsc.addupdate_scatter(tab, [didx + NP2], ea0_v[jj, sl])
                    plsc.addupdate_scatter(
                        tab, [didx + 2 * NP2], ea1_v[jj, sl])
                    plsc.addupdate_scatter(
                        tab, [didx + 3 * NP2], ea2_v[jj, sl])
            return carry

        lax.fori_loop(0, NCH // 8, window, 0)

        for c in range(4):
            pltpu.sync_copy(tab.at[pl.ds(c * NP2, NP2)],
                            part_sp.at[pl.ds(sid * NP2, NP2)])
            plsc.subcore_barrier()
            pltpu.sync_copy(part_sp.at[pl.ds(sid * RD, RD)], acc)
            for t in range(1, 16):
                pltpu.sync_copy(part_sp.at[pl.ds(t * NP2 + sid * RD, RD)], stg)

                def addb(r, carry):
                    sl = pl.ds(r * 16, 16)
                    acc[sl] = acc[sl] + stg[sl]
                    return carry

                lax.fori_loop(0, RD // 16, addb, 0, unroll=8)
            pltpu.sync_copy(
                acc, dm_hbm.at[pl.ds(cid * 4 * NP2 + c * NP2 + sid * RD, RD)])
            plsc.subcore_barrier()

    pl.run_scoped(
        body,
        pltpu.VMEM((8, CH), jnp.int32),
        pltpu.VMEM((8, CH), jnp.float32),
        pltpu.VMEM((8, CH), jnp.float32),
        pltpu.VMEM((8, CH), jnp.float32),
        pltpu.VMEM((4 * NP2,), jnp.float32),
    )


@functools.partial(
    pl.kernel,
    mesh=_MESH,
    out_type=[
        jax.ShapeDtypeStruct((2, NP, H), jnp.float32),
        jax.ShapeDtypeStruct((2 * NP2,), jnp.float32),
    ],
    scratch_types=[
        pltpu.VMEM((CH,), jnp.float32),
        pltpu.VMEM((128,), jnp.float32),
        pltpu.VMEM((128,), jnp.float32),
        pltpu.VMEM_SHARED((NP, H), jnp.float32),
        pltpu.VMEM_SHARED((16 * NP2,), jnp.float32),
        pltpu.SemaphoreType.DMA,
    ],
    compiler_params=_SC_PARAMS,
)
def _edge(h_hbm, pck_hbm, consts_hbm, src_hbm, dstae_hbm,
          num_hbm, den_hbm, alp_v, acc, stg, num_sp, dpart_sp, gsem):
    cid = lax.axis_index("c")
    sid = lax.axis_index("s")
    wid = cid * 16 + sid

    def body(pck_v, consts_v, src_v, dstae_v, cidx_v, rows, den_v):
        pltpu.sync_copy(pck_hbm, pck_v)
        pltpu.sync_copy(consts_hbm.at[3], consts_v)

        _zero_1d(den_v, NP)
        z16 = jnp.zeros((16,), jnp.float32)

        def zrow(r, carry):
            for c8 in range(H // 16):
                rows[r, pl.ds(c8 * 16, 16)] = z16
            return carry

        lax.fori_loop(0, CH, zrow, 0)
        r0 = sid * RPT
        for k in range(RPT // CH):
            pltpu.sync_copy(rows, num_sp.at[pl.ds(r0 + k * CH, CH)])
        rem = RPT - (RPT // CH) * CH
        if rem:
            pltpu.sync_copy(rows.at[pl.ds(0, rem)],
                            num_sp.at[pl.ds(r0 + RPT - rem, rem)])
        plsc.subcore_barrier()

        cvec = consts_v[pl.ds(48, 16)]
        mhi = jnp.full((16,), -65536, jnp.int32)
        mlo = jnp.full((16,), 65535, jnp.int32)

        def window(wi, carry):
            w0 = pl.multiple_of(wi * 8, 8)
            pltpu.sync_copy(src_hbm.at[wid, pl.ds(w0, 8)], src_v)
            pltpu.sync_copy(dstae_hbm.at[wid, pl.ds(w0, 8)], dstae_v)
            for jj in range(8):
                pltpu.async_copy(h_hbm.at[src_v.at[jj]], rows, gsem).wait()
                for g in range(CH // 16):
                    sl16 = pl.ds(g * 16, 16)
                    da = dstae_v[jj, sl16]
                    didx = da & mlo
                    cidx_v[sl16] = didx
                    ps = plsc.load_gather(pck_v, [src_v[jj, sl16]])
                    pd = plsc.load_gather(pck_v, [didx])
                    a_s = plsc.bitcast(ps & mhi, jnp.float32)
                    a_d = plsc.bitcast(lax.shift_left(pd, 16), jnp.float32)
                    a = a_s + a_d + plsc.bitcast(da & mhi, jnp.float32)
                    a = jnp.where(a >= 0, a, NEG * a)
                    al = jnp.exp(a - cvec)
                    alp_v[sl16] = al
                    plsc.addupdate_scatter(den_v, [didx], al)

                def sgrp(g2, carry2):
                    av = alp_v[pl.ds(g2 * 16, 16)]
                    for k in range(16):
                        s = av[k]
                        r = g2 * 16 + k
                        for c8 in range(H // 16):
                            csl = pl.ds(c8 * 16, 16)
                            rows[r, csl] = rows[r, csl] * s
                    return carry2

                lax.fori_loop(0, CH // 16, sgrp, 0)
                pltpu.sync_copy(rows, num_sp.at[cidx_v], add=True)
            return carry

        lax.fori_loop(0, NCH // 8, window, 0)
        pltpu.sync_copy(den_v, dpart_sp.at[pl.ds(sid * NP2, NP)])
        plsc.subcore_barrier()

        pltpu.sync_copy(num_sp.at[pl.ds(r0, RPT)],
                        num_hbm.at[cid, pl.ds(r0, RPT)])
        for b in range(RD // 128):
            o = sid * RD + b * 128
            pltpu.sync_copy(dpart_sp.at[pl.ds(o, 128)], acc)
            for t in range(1, 16):
                pltpu.sync_copy(dpart_sp.at[pl.ds(t * NP2 + o, 128)], stg)
                for r in range(8):
                    sl = pl.ds(r * 16, 16)
                    acc[sl] = acc[sl] + stg[sl]
            pltpu.sync_copy(acc, den_hbm.at[pl.ds(cid * NP2 + o, 128)])

    pl.run_scoped(
        body,
        pltpu.VMEM((NP,), jnp.int32),
        pltpu.VMEM((128,), jnp.float32),
        pltpu.VMEM((8, CH), jnp.int32),
        pltpu.VMEM((8, CH), jnp.int32),
        pltpu.VMEM((CH,), jnp.int32),
        pltpu.VMEM((CH, H), jnp.float32),
        pltpu.VMEM((NP,), jnp.float32),
    )


def _epi_body(num, h_in, asrc, adst, den_in, dm, consts, bgb, hout):
    nsum = num[0] + num[1]
    den = den_in[0, 0] + den_in[0, 1]
    h = h_in[...]
    deg = jnp.maximum(dm[0, 0, 0] + dm[0, 1, 0], 1.0)
    w3_0 = consts[0, 0]
    w3_1 = consts[1, 0]
    w3_2 = consts[2, 0]
    c = consts[3, 0]
    mlog = ((dm[0, 0, 1] + dm[0, 1, 1]) * w3_0
            + (dm[0, 0, 2] + dm[0, 1, 2]) * w3_1
            + (dm[0, 0, 3] + dm[0, 1, 3]) * w3_2) / deg
    logit = asrc[0, 0] + adst[0, 0] + mlog
    logit = jnp.where(logit >= 0, logit, NEG * logit)
    al = jnp.exp(logit - c)
    nsum = nsum + al[:, None] * h
    den = jnp.maximum(den + al, 1e-16)
    out = nsum / den[:, None] + bgb[0][None, :]
    out = jnp.where(out >= 0, out, ACT_NEG * out)
    m = jnp.mean(out, axis=1, keepdims=True)
    v = jnp.mean((out - m) ** 2, axis=1, keepdims=True)
    out_ln = (out - m) / jnp.sqrt(v + 1e-5) * bgb[1][None, :] + bgb[2][None, :]
    hout[...] = jnp.where(bgb[3, 0] > 0.5, out_ln, out)


def _epilogue(num, h, asrc, adst, den, dm, consts, bgb):
    return pl.pallas_call(
        _epi_body,
        grid=(NB,),
        in_specs=[
            pl.BlockSpec((2, NR, H), lambda i: (0, i, 0)),
            pl.BlockSpec((NR, H), lambda i: (i, 0)),
            pl.BlockSpec((1, 1, NR), lambda i: (i, 0, 0)),
            pl.BlockSpec((1, 1, NR), lambda i: (i, 0, 0)),
            pl.BlockSpec((1, 2, NR), lambda i: (i, 0, 0)),
            pl.BlockSpec((1, 2, 4, NR), lambda i: (i, 0, 0, 0)),
            pl.BlockSpec((8, 128), lambda i: (0, 0)),
            pl.BlockSpec((4, H), lambda i: (0, 0)),
        ],
        out_specs=pl.BlockSpec((NR, D), lambda i: (i, 0)),
        out_shape=jax.ShapeDtypeStruct((NP, D), jnp.float32),
    )(num, h, asrc, adst, den, dm, consts, bgb)


def _head_body(h, wm1, bm1, wm2, bm2, batchb, out, acc, cnt):
    i = pl.program_id(0)

    @pl.when(i == 0)
    def _():
        acc[...] = jnp.zeros_like(acc)
        cnt[...] = jnp.zeros_like(cnt)

    h2 = jnp.dot(h[...], wm1[...], preferred_element_type=jnp.float32) + bm1[...]
    h2 = jnp.dot(h2, wm2[...], preferred_element_type=jnp.float32) + bm2[...]
    b = batchb[0, 0]
    oh = (lax.broadcasted_iota(jnp.int32, (G, NR), 0) == b[None, :]).astype(
        jnp.float32)
    acc[...] = acc[...] + jnp.dot(oh, h2, preferred_element_type=jnp.float32)
    cnt[...] = cnt[...] + jnp.sum(oh, axis=1, keepdims=True)

    @pl.when(i == NB - 1)
    def _():
        out[...] = acc[...] / jnp.maximum(cnt[...], 1.0)


def _head(h, wm1, bm1, wm2, bm2, batchb):
    return pl.pallas_call(
        _head_body,
        grid=(NB,),
        in_specs=[
            pl.BlockSpec((NR, D), lambda i: (i, 0)),
            pl.BlockSpec((H, OUTD // 2), lambda i: (0, 0)),
            pl.BlockSpec((1, OUTD // 2), lambda i: (0, 0)),
            pl.BlockSpec((OUTD // 2, OUTD), lambda i: (0, 0)),
            pl.BlockSpec((1, OUTD), lambda i: (0, 0)),
            pl.BlockSpec((1, 1, NR), lambda i: (i, 0, 0)),
        ],
        out_specs=pl.BlockSpec((G, OUTD), lambda i: (0, 0)),
        out_shape=jax.ShapeDtypeStruct((G, OUTD), jnp.float32),
        scratch_shapes=[
            pltpu.VMEM((G, OUTD), jnp.float32),
            pltpu.VMEM((G, 128), jnp.float32),
        ],
    )(h, wm1, bm1, wm2, bm2, batchb)


def kernel(x, edge_index, edge_attr, batch, W0, as0, ad0, We0, ae0, b0,
           W1, as1, ad1, We1, ae1, b1, W2, as2, ad2, We2, ae2, b2,
           g0, be0, g1, be1, Wm1, bm1, Wm2, bm2):
    f32 = jnp.float32
    i32 = jnp.int32
    xp = jnp.zeros((NP, D), f32).at[:N, :].set(x)
    src = edge_index[0]
    dst = edge_index[1]
    padi = jnp.full((EP - E,), NP - 1, i32)
    src3 = jnp.concatenate([src, padi]).reshape(NT, NCH, CH)
    dst3 = jnp.concatenate([dst, padi]).reshape(NT, NCH, CH)
    ea_t = jnp.concatenate(
        [edge_attr, jnp.zeros((EP - E, 3), f32)], axis=0).T.reshape(
            3, NT, NCH, CH)
    batch3 = jnp.concatenate(
        [batch, jnp.full((NP - N,), G, i32)]).reshape(NB, 1, NR)

    dm = _phase0(dst3, ea_t).reshape(2, 4, NP2)[:, :, :NP].reshape(
        2, 4, NB, NR).transpose(2, 0, 1, 3)

    zeros_h = jnp.zeros((H,), f32)
    ones_h = jnp.ones((H,), f32)
    w_all = jnp.stack([W0, W1, W2])
    avec_all = jnp.stack([jnp.stack([as0, ad0]), jnp.stack([as1, ad1]),
                          jnp.stack([as2, ad2])])
    we_all = jnp.stack([We0, We1, We2])
    ae_all = jnp.stack([ae0.reshape(1, H), ae1.reshape(1, H),
                        ae2.reshape(1, H)])
    bgb_all = jnp.stack([
        jnp.stack([b0, g0, be0, ones_h]),
        jnp.stack([b1, g1, be1, ones_h]),
        jnp.stack([b2, zeros_h, zeros_h, zeros_h]),
    ])

    ea_r = ea_t.reshape(3, EP // 128, 128)

    dst_r = dst3.reshape(EP // 128, 128)

    def step(h, ws):
        w, avec, we, ae1, bgb = ws
        h_l, asrc, adst, pck, consts = _layer_a(h, w, avec, we, ae1)
        dstae = _aedge(ea_r, dst_r, we, ae1).reshape(NT, NCH, CH)
        num, den = _edge(h_l, pck.reshape(NP), consts, src3, dstae)
        den = den.reshape(2, NP2)[:, :NP].reshape(
            2, NB, NR).transpose(1, 0, 2)
        h_next = _epilogue(num, h_l, asrc, adst, den, dm, consts, bgb)
        return h_next, None

    h, _ = lax.scan(step, xp, (w_all, avec_all, we_all, ae_all, bgb_all))

    return _head(h, Wm1, bm1.reshape(1, OUTD // 2), Wm2,
                 bm2.reshape(1, OUTD), batch3)


# speedup vs baseline: 1.0787x; 1.0121x over previous
import functools

import jax
import jax.numpy as jnp
from jax import lax
from jax.experimental import pallas as pl
from jax.experimental.pallas import tpu as pltpu
from jax.experimental.pallas import tpu_sc as plsc

N = 10000
E = 320000
D = 128
H = 128
OUTD = 128
G = 64
NEG = 0.2
ACT_NEG = 0.01

NP = 10112
NT = 32
NB = NP // 128
NR = 128
NCH = 80
CH = 128
EPT = NCH * CH
EP = NT * EPT
RPT = NP // 16
NP2 = 10240
RD = NP2 // 16

_MESH = plsc.VectorSubcoreMesh(
    core_axis_name="c", subcore_axis_name="s", num_cores=2, num_subcores=16)
_SC_PARAMS = pltpu.CompilerParams(
    needs_layout_passes=False, skip_device_barrier=True,
    disable_bounds_checks=True, disable_semaphore_checks=True)


def _zero_1d(ref, n):
    z16 = jnp.zeros((16,), jnp.float32)

    def zb(r, carry):
        ref[pl.ds(r * 16, 16)] = z16
        return carry

    lax.fori_loop(0, n // 16, zb, 0, unroll=8)


def _layer_a_body(hin, w, avec, we, ae, h_o, asrc_o, adst_o, pck_o, consts_o,
                  macc):
    i = pl.program_id(0)
    h = jnp.dot(hin[...], w[...], preferred_element_type=jnp.float32)
    proj = jnp.dot(h, avec[...].T, preferred_element_type=jnp.float32)
    asr = proj[:, 0]
    adt = proj[:, 1]
    h_o[...] = h
    asrc_o[...] = asr[None, None, :]
    adst_o[...] = adt[None, None, :]
    ai = lax.bitcast_convert_type(asr, jnp.int32)
    di = lax.bitcast_convert_type(adt, jnp.int32)
    ai = (ai + 0x8000) & jnp.int32(-65536)
    di = lax.shift_right_logical((di + 0x8000) & jnp.int32(-65536), 16)
    pck_o[...] = (ai | di)[None, None, :]

    @pl.when(i == 0)
    def _():
        macc[0] = jnp.float32(-1e30)
        macc[1] = jnp.float32(-1e30)

    macc[0] = jnp.maximum(macc[0], jnp.max(asr))
    macc[1] = jnp.maximum(macc[1], jnp.max(adt))
    w3 = jnp.sum(we[...] * ae[...], axis=1)
    ub_e = jnp.sum(jnp.maximum(w3, 0.0))
    c = jnp.maximum(macc[0] + macc[1] + ub_e, 0.0)
    row = lax.broadcasted_iota(jnp.int32, (8, 128), 0)
    consts_o[...] = jnp.where(
        row == 0, w3[0], jnp.where(row == 1, w3[1], jnp.where(row == 2, w3[2], c)))


def _layer_a(hin, w, avec, we, ae1):
    return pl.pallas_call(
        _layer_a_body,
        grid=(NB,),
        in_specs=[
            pl.BlockSpec((NR, D), lambda i: (i, 0)),
            pl.BlockSpec((D, H), lambda i: (0, 0)),
            pl.BlockSpec((2, H), lambda i: (0, 0)),
            pl.BlockSpec((3, H), lambda i: (0, 0)),
            pl.BlockSpec((1, H), lambda i: (0, 0)),
        ],
        out_specs=[
            pl.BlockSpec((NR, H), lambda i: (i, 0)),
            pl.BlockSpec((1, 1, NR), lambda i: (i, 0, 0)),
            pl.BlockSpec((1, 1, NR), lambda i: (i, 0, 0)),
            pl.BlockSpec((1, 1, NR), lambda i: (i, 0, 0)),
            pl.BlockSpec((8, 128), lambda i: (0, 0)),
        ],
        out_shape=[
            jax.ShapeDtypeStruct((NP, H), jnp.float32),
            jax.ShapeDtypeStruct((NB, 1, NR), jnp.float32),
            jax.ShapeDtypeStruct((NB, 1, NR), jnp.float32),
            jax.ShapeDtypeStruct((NB, 1, NR), jnp.int32),
            jax.ShapeDtypeStruct((8, 128), jnp.float32),
        ],
        scratch_shapes=[pltpu.SMEM((2,), jnp.float32)],
    )(hin, w, avec, we, ae1)


def _aedge_body(ea, dst, we, ae, out):
    w3 = jnp.sum(we[...] * ae[...], axis=1)
    aev = ea[0] * w3[0] + ea[1] * w3[1] + ea[2] * w3[2]
    bits = (lax.bitcast_convert_type(aev, jnp.int32) + 0x8000) & jnp.int32(
        -65536)
    out[...] = bits | dst[...]


def _aedge(ea_r, dst_r, we, ae1):
    nblk = EP // (8 * 128)
    return pl.pallas_call(
        _aedge_body,
        grid=(nblk,),
        in_specs=[
            pl.BlockSpec((3, 8, 128), lambda i: (0, i, 0)),
            pl.BlockSpec((8, 128), lambda i: (i, 0)),
            pl.BlockSpec((3, H), lambda i: (0, 0)),
            pl.BlockSpec((1, H), lambda i: (0, 0)),
        ],
        out_specs=pl.BlockSpec((8, 128), lambda i: (i, 0)),
        out_shape=jax.ShapeDtypeStruct((EP // 128, 128), jnp.int32),
    )(ea_r, dst_r, we, ae1)


@functools.partial(
    pl.kernel,
    mesh=_MESH,
    out_type=jax.ShapeDtypeStruct((2 * 4 * NP2,), jnp.float32),
    scratch_types=[
        pltpu.VMEM((RD,), jnp.float32),
        pltpu.VMEM((RD,), jnp.float32),
        pltpu.VMEM_SHARED((16 * NP2,), jnp.float32),
    ],
    compiler_params=_SC_PARAMS,
)
def _phase0(dst_hbm, ea_hbm, dm_hbm, acc, stg, part_sp):
    cid = lax.axis_index("c")
    sid = lax.axis_index("s")
    wid = cid * 16 + sid

    def body(dst_v, ea0_v, ea1_v, ea2_v, tab):
        _zero_1d(tab, 4 * NP2)

        ones = jnp.ones((16,), jnp.float32)

        def window(wi, carry):
            w0 = pl.multiple_of(wi * 8, 8)
            pltpu.sync_copy(dst_hbm.at[wid, pl.ds(w0, 8)], dst_v)
            pltpu.sync_copy(ea_hbm.at[0, wid, pl.ds(w0, 8)], ea0_v)
            pltpu.sync_copy(ea_hbm.at[1, wid, pl.ds(w0, 8)], ea1_v)
            pltpu.sync_copy(ea_hbm.at[2, wid, pl.ds(w0, 8)], ea2_v)
            for jj in range(8):
                for g in range(CH // 16):
                    sl = pl.ds(g * 16, 16)
                    didx = dst_v[jj, sl]
                    plsc.addupdate_scatter(tab, [didx], ones)
                    plsc.addupdate_scatter(tab, [didx + NP2], ea0_v[jj, sl])
                    plsc.addupdate_scatter(
                        tab, [didx + 2 * NP2], ea1_v[jj, sl])
                    plsc.addupdate_scatter(
                        tab, [didx + 3 * NP2], ea2_v[jj, sl])
            return carry

        lax.fori_loop(0, NCH // 8, window, 0)

        for c in range(4):
            pltpu.sync_copy(tab.at[pl.ds(c * NP2, NP2)],
                            part_sp.at[pl.ds(sid * NP2, NP2)])
            plsc.subcore_barrier()
            pltpu.sync_copy(part_sp.at[pl.ds(sid * RD, RD)], acc)
            for t in range(1, 16):
                pltpu.sync_copy(part_sp.at[pl.ds(t * NP2 + sid * RD, RD)], stg)

                def addb(r, carry):
                    sl = pl.ds(r * 16, 16)
                    acc[sl] = acc[sl] + stg[sl]
                    return carry

                lax.fori_loop(0, RD // 16, addb, 0, unroll=8)
            pltpu.sync_copy(
                acc, dm_hbm.at[pl.ds(cid * 4 * NP2 + c * NP2 + sid * RD, RD)])
            plsc.subcore_barrier()

    pl.run_scoped(
        body,
        pltpu.VMEM((8, CH), jnp.int32),
        pltpu.VMEM((8, CH), jnp.float32),
        pltpu.VMEM((8, CH), jnp.float32),
        pltpu.VMEM((8, CH), jnp.float32),
        pltpu.VMEM((4 * NP2,), jnp.float32),
    )


@functools.partial(
    pl.kernel,
    mesh=_MESH,
    out_type=[
        jax.ShapeDtypeStruct((2, NP, H), jnp.float32),
        jax.ShapeDtypeStruct((2, 80, 128), jnp.float32),
    ],
    scratch_types=[
        pltpu.VMEM((CH,), jnp.float32),
        pltpu.VMEM_SHARED((NP, H), jnp.float32),
        pltpu.VMEM_SHARED((80, 128), jnp.float32),
        pltpu.SemaphoreType.DMA,
    ],
    compiler_params=_SC_PARAMS,
)
def _edge(h_hbm, pck_hbm, consts_hbm, src_hbm, dstae_hbm,
          num_hbm, den_hbm, alp_v, num_sp, den_sp, gsem):
    cid = lax.axis_index("c")
    sid = lax.axis_index("s")
    wid = cid * 16 + sid

    def body(pck_v, consts_v, src_v, dstae_v, cidx_v, idx80, rows, den_v):
        pltpu.sync_copy(pck_hbm, pck_v)
        pltpu.sync_copy(consts_hbm.at[3], consts_v)

        z16 = jnp.zeros((16,), jnp.float32)
        iota = lax.iota(jnp.int32, 16)
        for g in range(5):
            idx80[pl.ds(g * 16, 16)] = iota + g * 16

        def zden(r, carry):
            for c8 in range(H // 16):
                den_v[r, pl.ds(c8 * 16, 16)] = z16
            return carry

        lax.fori_loop(0, 80, zden, 0)

        def zrow(r, carry):
            for c8 in range(H // 16):
                rows[r, pl.ds(c8 * 16, 16)] = z16
            return carry

        lax.fori_loop(0, CH, zrow, 0)
        r0 = sid * RPT
        for k in range(RPT // CH):
            pltpu.sync_copy(rows, num_sp.at[pl.ds(r0 + k * CH, CH)])
        rem = RPT - (RPT // CH) * CH
        if rem:
            pltpu.sync_copy(rows.at[pl.ds(0, rem)],
                            num_sp.at[pl.ds(r0 + RPT - rem, rem)])

        @pl.when(sid < 10)
        def _():
            pltpu.sync_copy(rows.at[pl.ds(0, 8)],
                            den_sp.at[pl.ds(sid * 8, 8)])

        plsc.subcore_barrier()

        cvec = consts_v[pl.ds(48, 16)]
        mhi = jnp.full((16,), -65536, jnp.int32)
        mlo = jnp.full((16,), 65535, jnp.int32)
        m127 = jnp.full((16,), 127, jnp.int32)

        def window(wi, carry):
            w0 = pl.multiple_of(wi * 8, 8)
            pltpu.sync_copy(src_hbm.at[wid, pl.ds(w0, 8)], src_v)
            pltpu.sync_copy(dstae_hbm.at[wid, pl.ds(w0, 8)], dstae_v)
            for jj in range(8):
                pltpu.async_copy(h_hbm.at[src_v.at[jj]], rows, gsem).wait()
                for g in range(CH // 16):
                    sl16 = pl.ds(g * 16, 16)
                    da = dstae_v[jj, sl16]
                    didx = da & mlo
                    cidx_v[sl16] = didx
                    ps = plsc.load_gather(pck_v, [src_v[jj, sl16]])
                    pd = plsc.load_gather(pck_v, [didx])
                    a_s = plsc.bitcast(ps & mhi, jnp.float32)
                    a_d = plsc.bitcast(lax.shift_left(pd, 16), jnp.float32)
                    a = a_s + a_d + plsc.bitcast(da & mhi, jnp.float32)
                    a = jnp.where(a >= 0, a, NEG * a)
                    al = jnp.exp(a - cvec)
                    alp_v[sl16] = al
                    plsc.addupdate_scatter(
                        den_v,
                        [lax.shift_right_logical(didx, 7), didx & m127], al)

                def sgrp(g2, carry2):
                    av = alp_v[pl.ds(g2 * 16, 16)]
                    for k in range(16):
                        s = av[k]
                        r = g2 * 16 + k
                        for c8 in range(H // 16):
                            csl = pl.ds(c8 * 16, 16)
                            rows[r, csl] = rows[r, csl] * s
                    return carry2

                lax.fori_loop(0, CH // 16, sgrp, 0)
                pltpu.sync_copy(rows, num_sp.at[cidx_v], add=True)
            return carry

        lax.fori_loop(0, NCH // 8, window, 0)
        pltpu.sync_copy(den_v, den_sp.at[idx80], add=True)
        plsc.subcore_barrier()

        pltpu.sync_copy(num_sp.at[pl.ds(r0, RPT)],
                        num_hbm.at[cid, pl.ds(r0, RPT)])

        @pl.when(sid < 10)
        def _():
            pltpu.sync_copy(den_sp.at[pl.ds(sid * 8, 8)],
                            den_hbm.at[cid, pl.ds(sid * 8, 8)])

    pl.run_scoped(
        body,
        pltpu.VMEM((NP,), jnp.int32),
        pltpu.VMEM((128,), jnp.float32),
        pltpu.VMEM((8, CH), jnp.int32),
        pltpu.VMEM((8, CH), jnp.int32),
        pltpu.VMEM((CH,), jnp.int32),
        pltpu.VMEM((80,), jnp.int32),
        pltpu.VMEM((CH, H), jnp.float32),
        pltpu.VMEM((80, 128), jnp.float32),
    )


def _epi_body(num, h_in, asrc, adst, den_in, dm, consts, bgb, hout):
    nsum = num[0] + num[1]
    den = den_in[0, 0] + den_in[0, 1]
    h = h_in[...]
    deg = jnp.maximum(dm[0, 0, 0] + dm[0, 1, 0], 1.0)
    w3_0 = consts[0, 0]
    w3_1 = consts[1, 0]
    w3_2 = consts[2, 0]
    c = consts[3, 0]
    mlog = ((dm[0, 0, 1] + dm[0, 1, 1]) * w3_0
            + (dm[0, 0, 2] + dm[0, 1, 2]) * w3_1
            + (dm[0, 0, 3] + dm[0, 1, 3]) * w3_2) / deg
    logit = asrc[0, 0] + adst[0, 0] + mlog
    logit = jnp.where(logit >= 0, logit, NEG * logit)
    al = jnp.exp(logit - c)
    nsum = nsum + al[:, None] * h
    den = jnp.maximum(den + al, 1e-16)
    out = nsum / den[:, None] + bgb[0][None, :]
    out = jnp.where(out >= 0, out, ACT_NEG * out)
    m = jnp.mean(out, axis=1, keepdims=True)
    v = jnp.mean((out - m) ** 2, axis=1, keepdims=True)
    out_ln = (out - m) / jnp.sqrt(v + 1e-5) * bgb[1][None, :] + bgb[2][None, :]
    hout[...] = jnp.where(bgb[3, 0] > 0.5, out_ln, out)


def _epilogue(num, h, asrc, adst, den, dm, consts, bgb):
    return pl.pallas_call(
        _epi_body,
        grid=(NB,),
        in_specs=[
            pl.BlockSpec((2, NR, H), lambda i: (0, i, 0)),
            pl.BlockSpec((NR, H), lambda i: (i, 0)),
            pl.BlockSpec((1, 1, NR), lambda i: (i, 0, 0)),
            pl.BlockSpec((1, 1, NR), lambda i: (i, 0, 0)),
            pl.BlockSpec((1, 2, NR), lambda i: (i, 0, 0)),
            pl.BlockSpec((1, 2, 4, NR), lambda i: (i, 0, 0, 0)),
            pl.BlockSpec((8, 128), lambda i: (0, 0)),
            pl.BlockSpec((4, H), lambda i: (0, 0)),
        ],
        out_specs=pl.BlockSpec((NR, D), lambda i: (i, 0)),
        out_shape=jax.ShapeDtypeStruct((NP, D), jnp.float32),
    )(num, h, asrc, adst, den, dm, consts, bgb)


def _head_body(h, wm1, bm1, wm2, bm2, batchb, out, acc, cnt):
    i = pl.program_id(0)

    @pl.when(i == 0)
    def _():
        acc[...] = jnp.zeros_like(acc)
        cnt[...] = jnp.zeros_like(cnt)

    h2 = jnp.dot(h[...], wm1[...], preferred_element_type=jnp.float32) + bm1[...]
    h2 = jnp.dot(h2, wm2[...], preferred_element_type=jnp.float32) + bm2[...]
    b = batchb[0, 0]
    oh = (lax.broadcasted_iota(jnp.int32, (G, NR), 0) == b[None, :]).astype(
        jnp.float32)
    acc[...] = acc[...] + jnp.dot(oh, h2, preferred_element_type=jnp.float32)
    cnt[...] = cnt[...] + jnp.sum(oh, axis=1, keepdims=True)

    @pl.when(i == NB - 1)
    def _():
        out[...] = acc[...] / jnp.maximum(cnt[...], 1.0)


def _head(h, wm1, bm1, wm2, bm2, batchb):
    return pl.pallas_call(
        _head_body,
        grid=(NB,),
        in_specs=[
            pl.BlockSpec((NR, D), lambda i: (i, 0)),
            pl.BlockSpec((H, OUTD // 2), lambda i: (0, 0)),
            pl.BlockSpec((1, OUTD // 2), lambda i: (0, 0)),
            pl.BlockSpec((OUTD // 2, OUTD), lambda i: (0, 0)),
            pl.BlockSpec((1, OUTD), lambda i: (0, 0)),
            pl.BlockSpec((1, 1, NR), lambda i: (i, 0, 0)),
        ],
        out_specs=pl.BlockSpec((G, OUTD), lambda i: (0, 0)),
        out_shape=jax.ShapeDtypeStruct((G, OUTD), jnp.float32),
        scratch_shapes=[
            pltpu.VMEM((G, OUTD), jnp.float32),
            pltpu.VMEM((G, 128), jnp.float32),
        ],
    )(h, wm1, bm1, wm2, bm2, batchb)


def kernel(x, edge_index, edge_attr, batch, W0, as0, ad0, We0, ae0, b0,
           W1, as1, ad1, We1, ae1, b1, W2, as2, ad2, We2, ae2, b2,
           g0, be0, g1, be1, Wm1, bm1, Wm2, bm2):
    f32 = jnp.float32
    i32 = jnp.int32
    xp = jnp.zeros((NP, D), f32).at[:N, :].set(x)
    src = edge_index[0]
    dst = edge_index[1]
    padi = jnp.full((EP - E,), NP - 1, i32)
    src3 = jnp.concatenate([src, padi]).reshape(NT, NCH, CH)
    dst3 = jnp.concatenate([dst, padi]).reshape(NT, NCH, CH)
    ea_t = jnp.concatenate(
        [edge_attr, jnp.zeros((EP - E, 3), f32)], axis=0).T.reshape(
            3, NT, NCH, CH)
    batch3 = jnp.concatenate(
        [batch, jnp.full((NP - N,), G, i32)]).reshape(NB, 1, NR)

    dm = _phase0(dst3, ea_t).reshape(2, 4, NP2)[:, :, :NP].reshape(
        2, 4, NB, NR).transpose(2, 0, 1, 3)

    zeros_h = jnp.zeros((H,), f32)
    ones_h = jnp.ones((H,), f32)
    w_all = jnp.stack([W0, W1, W2])
    avec_all = jnp.stack([jnp.stack([as0, ad0]), jnp.stack([as1, ad1]),
                          jnp.stack([as2, ad2])])
    we_all = jnp.stack([We0, We1, We2])
    ae_all = jnp.stack([ae0.reshape(1, H), ae1.reshape(1, H),
                        ae2.reshape(1, H)])
    bgb_all = jnp.stack([
        jnp.stack([b0, g0, be0, ones_h]),
        jnp.stack([b1, g1, be1, ones_h]),
        jnp.stack([b2, zeros_h, zeros_h, zeros_h]),
    ])

    ea_r = ea_t.reshape(3, EP // 128, 128)

    dst_r = dst3.reshape(EP // 128, 128)

    def step(h, ws):
        w, avec, we, ae1, bgb = ws
        h_l, asrc, adst, pck, consts = _layer_a(h, w, avec, we, ae1)
        dstae = _aedge(ea_r, dst_r, we, ae1).reshape(NT, NCH, CH)
        num, den = _edge(h_l, pck.reshape(NP), consts, src3, dstae)
        den = den.reshape(2, NP2)[:, :NP].reshape(
            2, NB, NR).transpose(1, 0, 2)
        h_next = _epilogue(num, h_l, asrc, adst, den, dm, consts, bgb)
        return h_next, None

    h, _ = lax.scan(step, xp, (w_all, avec_all, we_all, ae_all, bgb_all))

    return _head(h, Wm1, bm1.reshape(1, OUTD // 2), Wm2,
                 bm2.reshape(1, OUTD), batch3)
